# baseline (speedup 1.0000x reference)
import jax
import jax.numpy as jnp
from jax.experimental import pallas as pl

T = 2048
D = 2048
E = 16
K = 8
F = 768

BT_R = 512
EG = 1


def _router_body(x_ref, wr_ref, comb_ref):
    logits = jnp.dot(x_ref[...], wr_ref[...], preferred_element_type=jnp.float32)
    p = jax.nn.softmax(logits, axis=-1)
    pw = p
    sel = jnp.zeros_like(p, dtype=jnp.bool_)
    col = jax.lax.broadcasted_iota(jnp.int32, p.shape, 1)
    for _ in range(K):
        idx = jnp.argmax(pw, axis=-1)
        oh = col == idx[:, None]
        sel = jnp.logical_or(sel, oh)
        pw = jnp.where(oh, -jnp.inf, pw)
    wsel = jnp.where(sel, p, 0.0)
    comb_ref[...] = wsel / jnp.sum(wsel, axis=-1, keepdims=True)


def _gateup_body(x_ref, wg_ref, wu_ref, comb_ref, h_ref):
    e = pl.program_id(0)
    xb = x_ref[...]
    g = jnp.dot(xb, wg_ref[0].astype(jnp.bfloat16),
                preferred_element_type=jnp.float32)
    u = jnp.dot(xb, wu_ref[0].astype(jnp.bfloat16),
                preferred_element_type=jnp.float32)
    lane = jax.lax.broadcasted_iota(jnp.int32, (1, E), 1)
    w = jnp.sum(jnp.where(lane == e, comb_ref[...], 0.0), axis=1, keepdims=True)
    h = g * jax.nn.sigmoid(g) * u * w
    h_ref[...] = h.astype(jnp.bfloat16)


def _down_body(h_ref, wd_ref, out_ref):
    g = pl.program_id(0)
    y = jnp.dot(h_ref[...], wd_ref[...].astype(jnp.bfloat16),
                preferred_element_type=jnp.float32)

    @pl.when(g == 0)
    def _():
        out_ref[...] = y

    @pl.when(g > 0)
    def _():
        out_ref[...] += y


def kernel(x, W_router, w_gate, w_up, w_down):
    combine = pl.pallas_call(
        _router_body,
        grid=(T // BT_R,),
        in_specs=[
            pl.BlockSpec((BT_R, D), lambda t: (t, 0)),
            pl.BlockSpec((D, E), lambda t: (0, 0)),
        ],
        out_specs=pl.BlockSpec((BT_R, E), lambda t: (t, 0)),
        out_shape=jax.ShapeDtypeStruct((T, E), jnp.float32),
    )(x, W_router)

    xb = x.astype(jnp.bfloat16)
    wg = w_gate
    wu = w_up
    wd = w_down.reshape(E * F, D)

    h = pl.pallas_call(
        _gateup_body,
        grid=(E,),
        in_specs=[
            pl.BlockSpec((T, D), lambda e: (0, 0)),
            pl.BlockSpec((1, D, F), lambda e: (e, 0, 0)),
            pl.BlockSpec((1, D, F), lambda e: (e, 0, 0)),
            pl.BlockSpec((T, E), lambda e: (0, 0)),
        ],
        out_specs=pl.BlockSpec((T, F), lambda e: (0, e)),
        out_shape=jax.ShapeDtypeStruct((T, E * F), jnp.bfloat16),
    )(xb, wg, wu, combine)

    out = pl.pallas_call(
        _down_body,
        grid=(E // EG,),
        in_specs=[
            pl.BlockSpec((T, EG * F), lambda g: (0, g)),
            pl.BlockSpec((EG * F, D), lambda g: (g, 0)),
        ],
        out_specs=pl.BlockSpec((T, D), lambda g: (0, 0)),
        out_shape=jax.ShapeDtypeStruct((T, D), jnp.float32),
    )(h, wd)
    return out


# speedup vs baseline: 1.0333x; 1.0333x over previous
import jax
import jax.numpy as jnp
from jax.experimental import pallas as pl

T = 2048
D = 2048
E = 16
K = 8
F = 768

BT_R = 512
EG = 2
BTB = 1024


def _router_body(x_ref, wr_ref, comb_ref):
    logits = jnp.dot(x_ref[...], wr_ref[...], preferred_element_type=jnp.float32)
    p = jax.nn.softmax(logits, axis=-1)
    pw = p
    sel = jnp.zeros_like(p, dtype=jnp.bool_)
    col = jax.lax.broadcasted_iota(jnp.int32, p.shape, 1)
    for _ in range(K):
        idx = jnp.argmax(pw, axis=-1)
        oh = col == idx[:, None]
        sel = jnp.logical_or(sel, oh)
        pw = jnp.where(oh, -jnp.inf, pw)
    wsel = jnp.where(sel, p, 0.0)
    comb_ref[...] = wsel / jnp.sum(wsel, axis=-1, keepdims=True)


def _gateup_body(x_ref, wg_ref, wu_ref, comb_ref, h_ref):
    e = pl.program_id(0)
    xb = x_ref[...]
    g = jnp.dot(xb, wg_ref[0].astype(jnp.bfloat16),
                preferred_element_type=jnp.float32)
    u = jnp.dot(xb, wu_ref[0].astype(jnp.bfloat16),
                preferred_element_type=jnp.float32)
    lane = jax.lax.broadcasted_iota(jnp.int32, (1, E), 1)
    w = jnp.sum(jnp.where(lane == e, comb_ref[...], 0.0), axis=1, keepdims=True)
    h = g * jax.nn.sigmoid(g) * u * w
    h_ref[...] = h.astype(jnp.bfloat16)


def _down_body(h_ref, wd_ref, out_ref):
    g = pl.program_id(1)
    y = jnp.dot(h_ref[...], wd_ref[...].astype(jnp.bfloat16),
                preferred_element_type=jnp.float32)

    @pl.when(g == 0)
    def _():
        out_ref[...] = y

    @pl.when(g > 0)
    def _():
        out_ref[...] += y


def kernel(x, W_router, w_gate, w_up, w_down):
    combine = pl.pallas_call(
        _router_body,
        grid=(T // BT_R,),
        in_specs=[
            pl.BlockSpec((BT_R, D), lambda t: (t, 0)),
            pl.BlockSpec((D, E), lambda t: (0, 0)),
        ],
        out_specs=pl.BlockSpec((BT_R, E), lambda t: (t, 0)),
        out_shape=jax.ShapeDtypeStruct((T, E), jnp.float32),
    )(x, W_router)

    xb = x.astype(jnp.bfloat16)
    wg = w_gate
    wu = w_up
    wd = w_down.reshape(E * F, D)

    h = pl.pallas_call(
        _gateup_body,
        grid=(E,),
        in_specs=[
            pl.BlockSpec((T, D), lambda e: (0, 0)),
            pl.BlockSpec((1, D, F), lambda e: (e, 0, 0)),
            pl.BlockSpec((1, D, F), lambda e: (e, 0, 0)),
            pl.BlockSpec((T, E), lambda e: (0, 0)),
        ],
        out_specs=pl.BlockSpec((T, F), lambda e: (0, e)),
        out_shape=jax.ShapeDtypeStruct((T, E * F), jnp.bfloat16),
    )(xb, wg, wu, combine)

    out = pl.pallas_call(
        _down_body,
        grid=(T // BTB, E // EG),
        in_specs=[
            pl.BlockSpec((BTB, EG * F), lambda t, g: (t, g)),
            pl.BlockSpec((EG * F, D), lambda t, g: (g, 0)),
        ],
        out_specs=pl.BlockSpec((BTB, D), lambda t, g: (t, 0)),
        out_shape=jax.ShapeDtypeStruct((T, D), jnp.float32),
    )(h, wd)
    return out


# speedup vs baseline: 1.0534x; 1.0195x over previous
import jax
import jax.numpy as jnp
from jax.experimental import pallas as pl

T = 2048
D = 2048
E = 16
K = 8
F = 768

BT_R = 512
EG = 2
BTB = 1024


def _router_body(x_ref, wr_ref, comb_ref, xb_ref):
    xb_ref[...] = x_ref[...].astype(jnp.bfloat16)
    logits = jnp.dot(x_ref[...], wr_ref[...], preferred_element_type=jnp.float32)
    p = jax.nn.softmax(logits, axis=-1)
    pw = p
    sel = jnp.zeros_like(p, dtype=jnp.bool_)
    col = jax.lax.broadcasted_iota(jnp.int32, p.shape, 1)
    for _ in range(K):
        idx = jnp.argmax(pw, axis=-1)
        oh = col == idx[:, None]
        sel = jnp.logical_or(sel, oh)
        pw = jnp.where(oh, -jnp.inf, pw)
    wsel = jnp.where(sel, p, 0.0)
    comb_ref[...] = wsel / jnp.sum(wsel, axis=-1, keepdims=True)


def _gateup_body(x_ref, wg_ref, wu_ref, comb_ref, h_ref):
    e = pl.program_id(0)
    xb = x_ref[...]
    g = jnp.dot(xb, wg_ref[0].astype(jnp.bfloat16),
                preferred_element_type=jnp.float32)
    u = jnp.dot(xb, wu_ref[0].astype(jnp.bfloat16),
                preferred_element_type=jnp.float32)
    lane = jax.lax.broadcasted_iota(jnp.int32, (1, E), 1)
    w = jnp.sum(jnp.where(lane == e, comb_ref[...], 0.0), axis=1, keepdims=True)
    h = g * jax.nn.sigmoid(g) * u * w
    h_ref[...] = h.astype(jnp.bfloat16)


def _down_body(h_ref, wd_ref, out_ref):
    g = pl.program_id(1)
    y = jnp.dot(h_ref[...], wd_ref[...].astype(jnp.bfloat16),
                preferred_element_type=jnp.float32)

    @pl.when(g == 0)
    def _():
        out_ref[...] = y

    @pl.when(g > 0)
    def _():
        out_ref[...] += y


def kernel(x, W_router, w_gate, w_up, w_down):
    combine, xb = pl.pallas_call(
        _router_body,
        grid=(T // BT_R,),
        in_specs=[
            pl.BlockSpec((BT_R, D), lambda t: (t, 0)),
            pl.BlockSpec((D, E), lambda t: (0, 0)),
        ],
        out_specs=[
            pl.BlockSpec((BT_R, E), lambda t: (t, 0)),
            pl.BlockSpec((BT_R, D), lambda t: (t, 0)),
        ],
        out_shape=[
            jax.ShapeDtypeStruct((T, E), jnp.float32),
            jax.ShapeDtypeStruct((T, D), jnp.bfloat16),
        ],
    )(x, W_router)

    wg = w_gate
    wu = w_up
    wd = w_down.reshape(E * F, D)

    h = pl.pallas_call(
        _gateup_body,
        grid=(E,),
        in_specs=[
            pl.BlockSpec((T, D), lambda e: (0, 0)),
            pl.BlockSpec((1, D, F), lambda e: (e, 0, 0)),
            pl.BlockSpec((1, D, F), lambda e: (e, 0, 0)),
            pl.BlockSpec((T, E), lambda e: (0, 0)),
        ],
        out_specs=pl.BlockSpec((T, F), lambda e: (0, e)),
        out_shape=jax.ShapeDtypeStruct((T, E * F), jnp.bfloat16),
    )(xb, wg, wu, combine)

    out = pl.pallas_call(
        _down_body,
        grid=(T // BTB, E // EG),
        in_specs=[
            pl.BlockSpec((BTB, EG * F), lambda t, g: (t, g)),
            pl.BlockSpec((EG * F, D), lambda t, g: (g, 0)),
        ],
        out_specs=pl.BlockSpec((BTB, D), lambda t, g: (t, 0)),
        out_shape=jax.ShapeDtypeStruct((T, D), jnp.float32),
    )(h, wd)
    return out
